# baseline (device time: 195381 ns/iter reference)
import jax
import jax.numpy as jnp
from jax import lax
from jax.experimental import pallas as pl
from jax.experimental.pallas import tpu as pltpu

M_TOTAL = 4096
M_OUT = 2048
HALF = 1024
D = 2048
EPS = 1e-6


def kernel(partial, gamma):
    x = partial.reshape(M_TOTAL, D)
    g = gamma.reshape(1, D)

    def body(x_ref, g_ref, out_ref, recv_buf, local_buf,
             local_sem, p1_send, p1_recv, p2_send, p2_recv):
        my_x = lax.axis_index("x")
        my_y = lax.axis_index("y")
        other_x = 1 - my_x
        other_y = 1 - my_y

        my_rows = my_x * M_OUT + my_y * HALF
        partner_rows = other_x * M_OUT + my_y * HALF

        local_cp = pltpu.make_async_copy(
            x_ref.at[pl.ds(my_rows, HALF), :], local_buf, local_sem)
        local_cp.start()

        barrier = pltpu.get_barrier_semaphore()
        pl.semaphore_signal(barrier, inc=1, device_id=(other_x, my_y),
                            device_id_type=pl.DeviceIdType.MESH)
        pl.semaphore_signal(barrier, inc=1, device_id=(my_x, other_y),
                            device_id_type=pl.DeviceIdType.MESH)
        pl.semaphore_wait(barrier, 2)

        p1 = pltpu.make_async_remote_copy(
            src_ref=x_ref.at[pl.ds(partner_rows, HALF), :],
            dst_ref=recv_buf,
            send_sem=p1_send, recv_sem=p1_recv,
            device_id=(other_x, my_y), device_id_type=pl.DeviceIdType.MESH,
        )
        p1.start()
        p1.wait()
        local_cp.wait()

        y = local_buf[...] + recv_buf[...]
        ms = jnp.mean(y * y, axis=-1, keepdims=True)
        out_ref[pl.ds(my_y * HALF, HALF), :] = (
            y * lax.rsqrt(ms + EPS) * g_ref[...])

        p2 = pltpu.make_async_remote_copy(
            src_ref=out_ref.at[pl.ds(my_y * HALF, HALF), :],
            dst_ref=out_ref.at[pl.ds(my_y * HALF, HALF), :],
            send_sem=p2_send, recv_sem=p2_recv,
            device_id=(my_x, other_y), device_id_type=pl.DeviceIdType.MESH,
        )
        p2.start()
        p2.wait()

    return pl.pallas_call(
        body,
        out_shape=jax.ShapeDtypeStruct((M_OUT, D), jnp.float32),
        in_specs=[
            pl.BlockSpec(memory_space=pl.ANY),
            pl.BlockSpec(memory_space=pltpu.VMEM),
        ],
        out_specs=pl.BlockSpec(memory_space=pltpu.VMEM),
        scratch_shapes=[
            pltpu.VMEM((HALF, D), jnp.float32),
            pltpu.VMEM((HALF, D), jnp.float32),
            pltpu.SemaphoreType.DMA,
            pltpu.SemaphoreType.DMA,
            pltpu.SemaphoreType.DMA,
            pltpu.SemaphoreType.DMA,
            pltpu.SemaphoreType.DMA,
        ],
        compiler_params=pltpu.CompilerParams(collective_id=0),
    )(x, g)


# device time: 114908 ns/iter; 1.7003x vs baseline; 1.7003x over previous
import jax
import jax.numpy as jnp
from jax import lax
from jax.experimental import pallas as pl
from jax.experimental.pallas import tpu as pltpu

M_TOTAL = 4096
M_OUT = 2048
HALF = 1024
D = 2048
EPS = 1e-6
C = 8
R = HALF // C


def kernel(partial, gamma):
    x = partial.reshape(M_TOTAL, D)
    g = gamma.reshape(1, D)

    def body(x_ref, g_ref, out_ref, recv_buf, local_buf,
             local_sem, p1_send, p1_recv, p2_send, p2_recv):
        my_x = lax.axis_index("x")
        my_y = lax.axis_index("y")
        other_x = 1 - my_x
        other_y = 1 - my_y

        my_rows = my_x * M_OUT + my_y * HALF
        partner_rows = other_x * M_OUT + my_y * HALF

        local_cp = pltpu.make_async_copy(
            x_ref.at[pl.ds(my_rows, HALF), :], local_buf, local_sem)
        local_cp.start()

        barrier = pltpu.get_barrier_semaphore()
        pl.semaphore_signal(barrier, inc=1, device_id=(other_x, my_y),
                            device_id_type=pl.DeviceIdType.MESH)
        pl.semaphore_signal(barrier, inc=1, device_id=(my_x, other_y),
                            device_id_type=pl.DeviceIdType.MESH)
        pl.semaphore_wait(barrier, 2)

        p1 = []
        for i in range(C):
            r = pltpu.make_async_remote_copy(
                src_ref=x_ref.at[pl.ds(partner_rows + i * R, R), :],
                dst_ref=recv_buf.at[pl.ds(i * R, R), :],
                send_sem=p1_send.at[i], recv_sem=p1_recv.at[i],
                device_id=(other_x, my_y),
                device_id_type=pl.DeviceIdType.MESH,
            )
            r.start()
            p1.append(r)
        local_cp.wait()

        p2 = []
        for i in range(C):
            p1[i].wait_recv()
            y = (local_buf[pl.ds(i * R, R), :]
                 + recv_buf[pl.ds(i * R, R), :])
            ms = jnp.mean(y * y, axis=-1, keepdims=True)
            out_ref[pl.ds(my_y * HALF + i * R, R), :] = (
                y * lax.rsqrt(ms + EPS) * g_ref[...])
            r = pltpu.make_async_remote_copy(
                src_ref=out_ref.at[pl.ds(my_y * HALF + i * R, R), :],
                dst_ref=out_ref.at[pl.ds(my_y * HALF + i * R, R), :],
                send_sem=p2_send.at[i], recv_sem=p2_recv.at[i],
                device_id=(my_x, other_y),
                device_id_type=pl.DeviceIdType.MESH,
            )
            r.start()
            p2.append(r)

        for i in range(C):
            p1[i].wait_send()
            p2[i].wait_send()
            p2[i].wait_recv()

    return pl.pallas_call(
        body,
        out_shape=jax.ShapeDtypeStruct((M_OUT, D), jnp.float32),
        in_specs=[
            pl.BlockSpec(memory_space=pl.ANY),
            pl.BlockSpec(memory_space=pltpu.VMEM),
        ],
        out_specs=pl.BlockSpec(memory_space=pltpu.VMEM),
        scratch_shapes=[
            pltpu.VMEM((HALF, D), jnp.float32),
            pltpu.VMEM((HALF, D), jnp.float32),
            pltpu.SemaphoreType.DMA,
            pltpu.SemaphoreType.DMA((C,)),
            pltpu.SemaphoreType.DMA((C,)),
            pltpu.SemaphoreType.DMA((C,)),
            pltpu.SemaphoreType.DMA((C,)),
        ],
        compiler_params=pltpu.CompilerParams(collective_id=0),
    )(x, g)


# device time: 109886 ns/iter; 1.7780x vs baseline; 1.0457x over previous
import jax
import jax.numpy as jnp
from jax import lax
from jax.experimental import pallas as pl
from jax.experimental.pallas import tpu as pltpu

M_TOTAL = 4096
M_OUT = 2048
HALF = 1024
D = 2048
EPS = 1e-6
C = 16
R = HALF // C


def kernel(partial, gamma):
    x = partial.reshape(M_TOTAL, D)
    g = gamma.reshape(1, D)

    def body(x_ref, g_ref, out_ref, recv_buf, local_buf,
             local_sem, p1_send, p1_recv, p2_send, p2_recv):
        my_x = lax.axis_index("x")
        my_y = lax.axis_index("y")
        other_x = 1 - my_x
        other_y = 1 - my_y

        my_rows = my_x * M_OUT + my_y * HALF
        partner_rows = other_x * M_OUT + my_y * HALF

        local_cp = pltpu.make_async_copy(
            x_ref.at[pl.ds(my_rows, HALF), :], local_buf, local_sem)
        local_cp.start()

        barrier = pltpu.get_barrier_semaphore()
        pl.semaphore_signal(barrier, inc=1, device_id=(other_x, my_y),
                            device_id_type=pl.DeviceIdType.MESH)
        pl.semaphore_signal(barrier, inc=1, device_id=(my_x, other_y),
                            device_id_type=pl.DeviceIdType.MESH)
        pl.semaphore_wait(barrier, 2)

        p1 = []
        for i in range(C):
            r = pltpu.make_async_remote_copy(
                src_ref=x_ref.at[pl.ds(partner_rows + i * R, R), :],
                dst_ref=recv_buf.at[pl.ds(i * R, R), :],
                send_sem=p1_send.at[i], recv_sem=p1_recv.at[i],
                device_id=(other_x, my_y),
                device_id_type=pl.DeviceIdType.MESH,
            )
            r.start()
            p1.append(r)
        local_cp.wait()

        p2 = []
        for i in range(C):
            p1[i].wait_recv()
            y = (local_buf[pl.ds(i * R, R), :]
                 + recv_buf[pl.ds(i * R, R), :])
            ms = jnp.mean(y * y, axis=-1, keepdims=True)
            out_ref[pl.ds(my_y * HALF + i * R, R), :] = (
                y * lax.rsqrt(ms + EPS) * g_ref[...])
            r = pltpu.make_async_remote_copy(
                src_ref=out_ref.at[pl.ds(my_y * HALF + i * R, R), :],
                dst_ref=out_ref.at[pl.ds(my_y * HALF + i * R, R), :],
                send_sem=p2_send.at[i], recv_sem=p2_recv.at[i],
                device_id=(my_x, other_y),
                device_id_type=pl.DeviceIdType.MESH,
            )
            r.start()
            p2.append(r)

        for i in range(C):
            p1[i].wait_send()
            p2[i].wait_send()
            p2[i].wait_recv()

    return pl.pallas_call(
        body,
        out_shape=jax.ShapeDtypeStruct((M_OUT, D), jnp.float32),
        in_specs=[
            pl.BlockSpec(memory_space=pl.ANY),
            pl.BlockSpec(memory_space=pltpu.VMEM),
        ],
        out_specs=pl.BlockSpec(memory_space=pltpu.VMEM),
        scratch_shapes=[
            pltpu.VMEM((HALF, D), jnp.float32),
            pltpu.VMEM((HALF, D), jnp.float32),
            pltpu.SemaphoreType.DMA,
            pltpu.SemaphoreType.DMA((C,)),
            pltpu.SemaphoreType.DMA((C,)),
            pltpu.SemaphoreType.DMA((C,)),
            pltpu.SemaphoreType.DMA((C,)),
        ],
        compiler_params=pltpu.CompilerParams(collective_id=0),
    )(x, g)


# device time: 107193 ns/iter; 1.8227x vs baseline; 1.0251x over previous
import jax
import jax.numpy as jnp
from jax import lax
from jax.experimental import pallas as pl
from jax.experimental.pallas import tpu as pltpu

M_TOTAL = 4096
M_OUT = 2048
HALF = 1024
D = 2048
EPS = 1e-6
C = 32
R = HALF // C


def kernel(partial, gamma):
    x = partial.reshape(M_TOTAL, D)
    g = gamma.reshape(1, D)

    def body(x_ref, g_ref, out_ref, recv_buf, local_buf,
             local_sem, p1_send, p1_recv, p2_send, p2_recv):
        my_x = lax.axis_index("x")
        my_y = lax.axis_index("y")
        other_x = 1 - my_x
        other_y = 1 - my_y

        my_rows = my_x * M_OUT + my_y * HALF
        partner_rows = other_x * M_OUT + my_y * HALF

        local_cp = pltpu.make_async_copy(
            x_ref.at[pl.ds(my_rows, HALF), :], local_buf, local_sem)
        local_cp.start()

        barrier = pltpu.get_barrier_semaphore()
        pl.semaphore_signal(barrier, inc=1, device_id=(other_x, my_y),
                            device_id_type=pl.DeviceIdType.MESH)
        pl.semaphore_signal(barrier, inc=1, device_id=(my_x, other_y),
                            device_id_type=pl.DeviceIdType.MESH)
        pl.semaphore_wait(barrier, 2)

        p1 = []
        for i in range(C):
            r = pltpu.make_async_remote_copy(
                src_ref=x_ref.at[pl.ds(partner_rows + i * R, R), :],
                dst_ref=recv_buf.at[pl.ds(i * R, R), :],
                send_sem=p1_send.at[i], recv_sem=p1_recv.at[i],
                device_id=(other_x, my_y),
                device_id_type=pl.DeviceIdType.MESH,
            )
            r.start()
            p1.append(r)
        local_cp.wait()

        p2 = []
        for i in range(C):
            p1[i].wait_recv()
            y = (local_buf[pl.ds(i * R, R), :]
                 + recv_buf[pl.ds(i * R, R), :])
            ms = jnp.mean(y * y, axis=-1, keepdims=True)
            out_ref[pl.ds(my_y * HALF + i * R, R), :] = (
                y * lax.rsqrt(ms + EPS) * g_ref[...])
            r = pltpu.make_async_remote_copy(
                src_ref=out_ref.at[pl.ds(my_y * HALF + i * R, R), :],
                dst_ref=out_ref.at[pl.ds(my_y * HALF + i * R, R), :],
                send_sem=p2_send.at[i], recv_sem=p2_recv.at[i],
                device_id=(my_x, other_y),
                device_id_type=pl.DeviceIdType.MESH,
            )
            r.start()
            p2.append(r)

        for i in range(C):
            p1[i].wait_send()
            p2[i].wait_send()
            p2[i].wait_recv()

    return pl.pallas_call(
        body,
        out_shape=jax.ShapeDtypeStruct((M_OUT, D), jnp.float32),
        in_specs=[
            pl.BlockSpec(memory_space=pl.ANY),
            pl.BlockSpec(memory_space=pltpu.VMEM),
        ],
        out_specs=pl.BlockSpec(memory_space=pltpu.VMEM),
        scratch_shapes=[
            pltpu.VMEM((HALF, D), jnp.float32),
            pltpu.VMEM((HALF, D), jnp.float32),
            pltpu.SemaphoreType.DMA,
            pltpu.SemaphoreType.DMA((C,)),
            pltpu.SemaphoreType.DMA((C,)),
            pltpu.SemaphoreType.DMA((C,)),
            pltpu.SemaphoreType.DMA((C,)),
        ],
        compiler_params=pltpu.CompilerParams(collective_id=0),
    )(x, g)
